# baseline (speedup 1.0000x reference)
import jax
import jax.numpy as jnp
from jax import lax
from jax.experimental import pallas as pl
from jax.experimental.pallas import tpu as pltpu
from jax.experimental.pallas import tpu_sc as plsc

B = 4
S = 8192
D = 1024

_INFO = plsc.get_sparse_core_info()
NC = _INFO.num_cores
NS = _INFO.num_subcores
NW = NC * NS
LANES = 16

S_SC = 2048
S_TC = S - S_SC
ROWS_PER_W = S_SC // NW
C = 32
N_CHUNKS = ROWS_PER_W // C
TC_CS = 512


def _add_chunk(iobuf, tbuf):
    @plsc.parallel_loop(0, C, step=1)
    def _add_row(r):
        @plsc.parallel_loop(0, D, step=LANES, unroll=8)
        def _add(j):
            plsc.addupdate(iobuf.at[r, pl.ds(j, LANES)], tbuf[r, pl.ds(j, LANES)])


def _sc_body(in_hbm, tab_hbm, out_hbm, tbuf, io0, io1, sem_in, sem_out, sem_tab):
    cid = lax.axis_index("c")
    sid = lax.axis_index("s")
    wid = sid * NC + cid
    w_row0 = wid * ROWS_PER_W

    ios = (io0, io1)
    steps = [(c, b) for c in range(N_CHUNKS) for b in range(B)]
    n = len(steps)

    def in_copy(s, buf):
        c, b = steps[s]
        r0 = S_TC + w_row0 + c * C
        return pltpu.async_copy(in_hbm.at[b, pl.ds(r0, C)], buf, sem_in)

    tab_dma = pltpu.async_copy(tab_hbm.at[pl.ds(S_TC + w_row0, C)], tbuf, sem_tab)
    in_dma = in_copy(0, ios[0])
    out_pending = [None, None]

    for s, (c, b) in enumerate(steps):
        p = s % 2
        if b == 0:
            tab_dma.wait()
        if s + 1 < n:
            if out_pending[1 - p] is not None:
                out_pending[1 - p].wait()
            nxt = in_copy(s + 1, ios[1 - p])
        in_dma.wait()
        in_dma = nxt if s + 1 < n else None
        _add_chunk(ios[p], tbuf)
        if b == B - 1 and c + 1 < N_CHUNKS:
            tab_dma = pltpu.async_copy(
                tab_hbm.at[pl.ds(S_TC + w_row0 + (c + 1) * C, C)], tbuf, sem_tab
            )
        r0 = w_row0 + c * C
        out_pending[p] = pltpu.async_copy(ios[p], out_hbm.at[b, pl.ds(r0, C)], sem_out)

    for d in out_pending:
        if d is not None:
            d.wait()


TCB = 512
TC_NB = 6
N_TC_CHUNKS = S_TC // TCB


def _tc_body(tab_hbm, in_hbm, out_hbm, t0, t1, *rest):
    tabs = (t0, t1)
    ios = rest[:TC_NB]
    sem_in, sem_out, sem_tab = rest[TC_NB:]
    steps = [(c, b) for c in range(N_TC_CHUNKS) for b in range(B)]
    n = len(steps)

    def in_copy(s, buf):
        c, b = steps[s]
        return pltpu.make_async_copy(
            in_hbm.at[b, pl.ds(c * TCB, TCB)], buf, sem_in
        )

    def tab_copy(c):
        return pltpu.make_async_copy(
            tab_hbm.at[pl.ds(c * TCB, TCB)], tabs[c % 2], sem_tab
        )

    tab_pend = {}
    for c in range(min(2, N_TC_CHUNKS)):
        d = tab_copy(c)
        d.start()
        tab_pend[c] = d
    in_pend = {}
    for s in range(min(TC_NB - 1, n)):
        d = in_copy(s, ios[s % TC_NB])
        d.start()
        in_pend[s] = d
    out_pend = [None] * TC_NB

    for s, (c, b) in enumerate(steps):
        p = s % TC_NB
        if b == 0:
            tab_pend.pop(c).wait()
        nxt = s + TC_NB - 1
        if nxt < n:
            q = nxt % TC_NB
            if out_pend[q] is not None:
                out_pend[q].wait()
                out_pend[q] = None
            d = in_copy(nxt, ios[q])
            d.start()
            in_pend[nxt] = d
        in_pend.pop(s).wait()
        ios[p][...] = ios[p][...] + tabs[c % 2][...]
        if b == B - 1 and c + 2 < N_TC_CHUNKS:
            d = tab_copy(c + 2)
            d.start()
            tab_pend[c + 2] = d
        d = pltpu.make_async_copy(
            ios[p], out_hbm.at[b, pl.ds(c * TCB, TCB)], sem_out
        )
        d.start()
        out_pend[p] = d

    for d in out_pend:
        if d is not None:
            d.wait()


@jax.jit
def kernel(input_embeddings, emb_table):
    kfn = pl.kernel(
        _sc_body,
        out_type=jax.ShapeDtypeStruct((B, S_SC, D), jnp.float32),
        mesh=plsc.VectorSubcoreMesh(core_axis_name="c", subcore_axis_name="s"),
        scratch_types=[
            pltpu.VMEM((C, D), jnp.float32),
            pltpu.VMEM((C, D), jnp.float32),
            pltpu.VMEM((C, D), jnp.float32),
            pltpu.SemaphoreType.DMA,
            pltpu.SemaphoreType.DMA,
            pltpu.SemaphoreType.DMA,
        ],
    )
    sc_out = kfn(input_embeddings, emb_table)

    tc_full = pl.pallas_call(
        _tc_body,
        in_specs=[
            pl.BlockSpec(memory_space=pl.ANY),
            pl.BlockSpec(memory_space=pl.ANY),
        ],
        out_specs=pl.BlockSpec(memory_space=pl.ANY),
        out_shape=jax.ShapeDtypeStruct((B, S, D), jnp.float32),
        scratch_shapes=(
            [pltpu.VMEM((TCB, D), jnp.float32)] * (2 + TC_NB)
            + [pltpu.SemaphoreType.DMA] * 3
        ),
    )(emb_table, input_embeddings)

    sc_out, tc_full = jax.lax.optimization_barrier((sc_out, tc_full))
    return jax.lax.dynamic_update_slice(tc_full, sc_out, (0, S_TC, 0))


# speedup vs baseline: 1.0610x; 1.0610x over previous
import jax
import jax.numpy as jnp
from jax import lax
from jax.experimental import pallas as pl
from jax.experimental.pallas import tpu as pltpu
from jax.experimental.pallas import tpu_sc as plsc

B = 4
S = 8192
D = 1024

_INFO = plsc.get_sparse_core_info()
NC = _INFO.num_cores
NS = _INFO.num_subcores
NW = NC * NS
LANES = 16

S_SC = 2048
S_TC = S - S_SC
ROWS_PER_W = S_SC // NW
C = 32
N_CHUNKS = ROWS_PER_W // C
TC_CS = 2048


def _add_chunk(iobuf, tbuf):
    @plsc.parallel_loop(0, C, step=1)
    def _add_row(r):
        @plsc.parallel_loop(0, D, step=LANES, unroll=8)
        def _add(j):
            plsc.addupdate(iobuf.at[r, pl.ds(j, LANES)], tbuf[r, pl.ds(j, LANES)])


def _sc_body(in_hbm, tab_hbm, out_hbm, tbuf, io0, io1, sem_in, sem_out, sem_tab):
    cid = lax.axis_index("c")
    sid = lax.axis_index("s")
    wid = sid * NC + cid
    w_row0 = wid * ROWS_PER_W

    ios = (io0, io1)
    steps = [(c, b) for c in range(N_CHUNKS) for b in range(B)]
    n = len(steps)

    def in_copy(s, buf):
        c, b = steps[s]
        r0 = S_TC + w_row0 + c * C
        return pltpu.async_copy(in_hbm.at[b, pl.ds(r0, C)], buf, sem_in)

    tab_dma = pltpu.async_copy(tab_hbm.at[pl.ds(S_TC + w_row0, C)], tbuf, sem_tab)
    in_dma = in_copy(0, ios[0])
    out_pending = [None, None]

    for s, (c, b) in enumerate(steps):
        p = s % 2
        if b == 0:
            tab_dma.wait()
        if s + 1 < n:
            if out_pending[1 - p] is not None:
                out_pending[1 - p].wait()
            nxt = in_copy(s + 1, ios[1 - p])
        in_dma.wait()
        in_dma = nxt if s + 1 < n else None
        _add_chunk(ios[p], tbuf)
        if b == B - 1 and c + 1 < N_CHUNKS:
            tab_dma = pltpu.async_copy(
                tab_hbm.at[pl.ds(S_TC + w_row0 + (c + 1) * C, C)], tbuf, sem_tab
            )
        r0 = w_row0 + c * C
        out_pending[p] = pltpu.async_copy(ios[p], out_hbm.at[b, pl.ds(r0, C)], sem_out)

    for d in out_pending:
        if d is not None:
            d.wait()


def _tc_body(tab_ref, in_ref, out_ref):
    out_ref[...] = in_ref[...] + tab_ref[...][None]


@jax.jit
def kernel(input_embeddings, emb_table):
    kfn = pl.kernel(
        _sc_body,
        out_type=jax.ShapeDtypeStruct((B, S_SC, D), jnp.float32),
        mesh=plsc.VectorSubcoreMesh(core_axis_name="c", subcore_axis_name="s"),
        scratch_types=[
            pltpu.VMEM((C, D), jnp.float32),
            pltpu.VMEM((C, D), jnp.float32),
            pltpu.VMEM((C, D), jnp.float32),
            pltpu.SemaphoreType.DMA,
            pltpu.SemaphoreType.DMA,
            pltpu.SemaphoreType.DMA,
        ],
    )
    sc_out = kfn(input_embeddings, emb_table)

    tc_full = pl.pallas_call(
        _tc_body,
        grid=(S_TC // TC_CS, B),
        in_specs=[
            pl.BlockSpec((TC_CS, D), lambda s, b: (s, 0)),
            pl.BlockSpec((1, TC_CS, D), lambda s, b: (b, s, 0)),
        ],
        out_specs=pl.BlockSpec((1, TC_CS, D), lambda s, b: (b, s, 0)),
        out_shape=jax.ShapeDtypeStruct((B, S, D), jnp.float32),
    )(emb_table, input_embeddings)

    sc_out, tc_full = jax.lax.optimization_barrier((sc_out, tc_full))
    return jax.lax.dynamic_update_slice(tc_full, sc_out, (0, S_TC, 0))
